# baseline (speedup 1.0000x reference)
import functools

import jax
import jax.numpy as jnp
from jax import lax
from jax.experimental import pallas as pl
from jax.experimental.pallas import tpu as pltpu
from jax.experimental.pallas import tpu_sc as plsc

V = 10000
E = 320000
F = 128
K = 3
OUT = 128

NC = 2
NS = 16
NW = NC * NS
EPW = E // NW
C = 80
NCH = EPW // C
NSB = 5
NCHS = NCH // NSB
SBE = NCHS * C
VP = 10240
RPT = VP // NS
ZR = 64

_mesh = plsc.VectorSubcoreMesh(core_axis_name="c", subcore_axis_name="s")


def _seg_kernel_weighted(x_hbm, src_hbm, dst_hbm, lv_hbm, out_hbm,
                         src_v, dst_v, lv_v, rows_v, zb_v, acc,
                         g0, g1, s0, s1):
    _seg_body(x_hbm, src_hbm, dst_hbm, lv_hbm, out_hbm,
              src_v, dst_v, lv_v, rows_v, zb_v, acc,
              (g0, g1), (s0, s1), weighted=True)


def _seg_kernel_plain(x_hbm, src_hbm, dst_hbm, out_hbm,
                      src_v, dst_v, rows_v, zb_v, acc,
                      g0, g1, s0, s1):
    _seg_body(x_hbm, src_hbm, dst_hbm, None, out_hbm,
              src_v, dst_v, None, rows_v, zb_v, acc,
              (g0, g1), (s0, s1), weighted=False)


def _seg_body(x_hbm, src_hbm, dst_hbm, lv_hbm, out_hbm,
              src_v, dst_v, lv_v, rows_v, zb_v, acc, gsems, ssems, *,
              weighted):
    cid = lax.axis_index("c")
    sid = lax.axis_index("s")
    w = cid * NS + sid

    def _zrow(i, carry):
        for r in range(F // 16):
            zb_v[i, pl.ds(r * 16, 16)] = jnp.zeros((16,), jnp.float32)
        return carry

    lax.fori_loop(0, ZR, _zrow, 0)
    zcopies = [
        pltpu.async_copy(zb_v, acc.at[pl.ds(sid * RPT + i * ZR, ZR)],
                         ssems[0])
        for i in range(RPT // ZR)
    ]

    def _g_start(j, slot):
        pltpu.async_copy(x_hbm.at[src_v.at[j]], rows_v.at[slot], gsems[slot])

    def _g_wait(j, slot):
        pltpu.make_async_copy(x_hbm.at[src_v.at[j]], rows_v.at[slot],
                              gsems[slot]).wait()

    def _s_start(j, slot):
        pltpu.async_copy(rows_v.at[slot], acc.at[dst_v.at[j]], ssems[slot],
                         add=True)

    def _s_wait(j, slot):
        pltpu.make_async_copy(rows_v.at[slot], acc.at[dst_v.at[j]],
                              ssems[slot]).wait()

    def _scale(j, slot):
        def _group(g, carry):
            lvec = lv_v[j, pl.ds(g * 16, 16)]
            for t in range(16):
                e = g * 16 + t
                sv = jnp.broadcast_to(lvec[t], (16,))
                for r in range(F // 16):
                    sl = pl.ds(r * 16, 16)
                    rows_v[slot, e, sl] = rows_v[slot, e, sl] * sv
            return carry
        lax.fori_loop(0, C // 16, _group, 0)

    def _process(j, slot, start_next, wait_prev_scatter):
        _g_wait(j, slot)
        if weighted:
            _scale(j, slot)
        _s_start(j, slot)
        if start_next:
            if wait_prev_scatter:
                _s_wait(j - 1, 1 - slot)
            _g_start(j + 1, 1 - slot)

    def _pair(p, carry):
        j = 2 * p + 1
        _process(j, 1, True, True)
        _process(j + 1, 0, True, True)
        return carry

    for sb in range(NSB):
        pltpu.sync_copy(src_hbm.at[w, sb], src_v)
        pltpu.sync_copy(dst_hbm.at[w, sb], dst_v)
        if weighted:
            pltpu.sync_copy(lv_hbm.at[w, sb], lv_v)
        _g_start(0, 0)
        if sb == 0:
            for zc in zcopies:
                zc.wait()
            plsc.subcore_barrier()
        _process(0, 0, True, False)
        lax.fori_loop(0, (NCHS - 3) // 2, _pair, 0)
        _process(NCHS - 2, 1, True, True)
        _process(NCHS - 1, 0, False, False)
        _s_wait(NCHS - 2, 1)
        _s_wait(NCHS - 1, 0)

    plsc.subcore_barrier()
    pltpu.sync_copy(acc.at[pl.ds(sid * RPT, RPT)],
                    out_hbm.at[cid, pl.ds(sid * RPT, RPT)])


def _make_seg(weighted):
    body = _seg_kernel_weighted if weighted else _seg_kernel_plain
    scratch = [
        pltpu.VMEM((NCHS, C), jnp.int32),
        pltpu.VMEM((NCHS, C), jnp.int32),
    ]
    if weighted:
        scratch.append(pltpu.VMEM((NCHS, C), jnp.float32))
    scratch += [
        pltpu.VMEM((2, C, F), jnp.float32),
        pltpu.VMEM((ZR, F), jnp.float32),
        pltpu.VMEM_SHARED((VP, F), jnp.float32),
        pltpu.SemaphoreType.DMA,
        pltpu.SemaphoreType.DMA,
        pltpu.SemaphoreType.DMA,
        pltpu.SemaphoreType.DMA,
    ]
    return pl.kernel(
        body,
        out_type=jax.ShapeDtypeStruct((NC, VP, F), jnp.float32),
        mesh=_mesh,
        scratch_types=scratch,
    )


_seg_weighted = _make_seg(True)
_seg_plain = _make_seg(False)


_VB = 400


def _combine1_body(p0, p1, o):
    o[...] = p0[...] + p1[...]


def _combine2_body(p0, p1, x, o):
    o[...] = 2.0 * (p0[...] + p1[...]) - x[...]


def _relu_body(p0, p1, b, o):
    o[...] = jnp.maximum(p0[...] + p1[...] + b[...], 0.0)


def _matmul_body(h, w, o):
    o[...] = jnp.dot(h[...], w[...], preferred_element_type=jnp.float32)


def _ew_spec():
    return pl.BlockSpec((_VB, F), lambda i: (i, 0))


def _combine1(p0, p1):
    return pl.pallas_call(
        _combine1_body,
        grid=(V // _VB,),
        in_specs=[_ew_spec(), _ew_spec()],
        out_specs=_ew_spec(),
        out_shape=jax.ShapeDtypeStruct((V, F), jnp.float32),
    )(p0, p1)


def _combine2(p0, p1, x):
    return pl.pallas_call(
        _combine2_body,
        grid=(V // _VB,),
        in_specs=[_ew_spec(), _ew_spec(), _ew_spec()],
        out_specs=_ew_spec(),
        out_shape=jax.ShapeDtypeStruct((V, F), jnp.float32),
    )(p0, p1, x)


def _relu_out(p0, p1, b2):
    return pl.pallas_call(
        _relu_body,
        grid=(V // _VB,),
        in_specs=[_ew_spec(), _ew_spec(),
                  pl.BlockSpec((1, F), lambda i: (0, 0))],
        out_specs=_ew_spec(),
        out_shape=jax.ShapeDtypeStruct((V, OUT), jnp.float32),
    )(p0, p1, b2)


def _matmul(h, w):
    return pl.pallas_call(
        _matmul_body,
        grid=(V // _VB,),
        in_specs=[pl.BlockSpec((_VB, F * K), lambda i: (i, 0)),
                  pl.BlockSpec((F * K, OUT), lambda i: (0, 0))],
        out_specs=pl.BlockSpec((_VB, OUT), lambda i: (i, 0)),
        out_shape=jax.ShapeDtypeStruct((V, OUT), jnp.float32),
    )(h, w)


def kernel(feature, edge_index, L_values, W, b):
    src = edge_index[0].reshape(NW, NSB, NCHS, C)
    dst = edge_index[1].reshape(NW, NSB, NCHS, C)
    lv = L_values.reshape(NW, NSB, NCHS, C)

    p1 = _seg_weighted(feature, src, dst, lv)
    t1 = _combine1(p1[0, :V], p1[1, :V])

    p2 = _seg_weighted(t1, src, dst, lv)
    t2 = _combine2(p2[0, :V], p2[1, :V], feature)

    tst = jnp.stack([feature, t1, t2], axis=0)
    h = jnp.transpose(tst, (2, 0, 1)).reshape(V, F * K)

    hw = _matmul(h, W)

    p3 = _seg_plain(hw, src, dst)
    return _relu_out(p3[0, :V], p3[1, :V], b.reshape(1, OUT))


# speedup vs baseline: 1.2842x; 1.2842x over previous
import functools

import jax
import jax.numpy as jnp
from jax import lax
from jax.experimental import pallas as pl
from jax.experimental.pallas import tpu as pltpu
from jax.experimental.pallas import tpu_sc as plsc

V = 10000
E = 320000
F = 128
K = 3
OUT = 128

NC = 2
NS = 16
NW = NC * NS
EPW = E // NW
C = 80
NCH = EPW // C
NSB = 5
NCHS = NCH // NSB
SBE = NCHS * C
VP = 10240
RPT = VP // NS
ZR = 64

_mesh = plsc.VectorSubcoreMesh(core_axis_name="c", subcore_axis_name="s")


def _seg_kernel_weighted(x_hbm, src_hbm, dst_hbm, lv_hbm, out_hbm,
                         src_v, lv_v, db0, db1, rows_v, zb_v, acc,
                         g0, g1, s0, s1):
    _seg_body(x_hbm, src_hbm, dst_hbm, lv_hbm, out_hbm,
              src_v, lv_v, (db0, db1), rows_v, zb_v, acc,
              (g0, g1), (s0, s1), weighted=True)


def _seg_kernel_plain(x_hbm, src_hbm, dst_hbm, out_hbm,
                      src_v, db0, db1, rows_v, zb_v, acc,
                      g0, g1, s0, s1):
    _seg_body(x_hbm, src_hbm, dst_hbm, None, out_hbm,
              src_v, None, (db0, db1), rows_v, zb_v, acc,
              (g0, g1), (s0, s1), weighted=False)


def _seg_body(x_hbm, src_hbm, dst_hbm, lv_hbm, out_hbm,
              src_v, lv_v, dstbs, rows_v, zb_v, acc, gsems, ssems, *,
              weighted):
    cid = lax.axis_index("c")
    sid = lax.axis_index("s")
    w = cid * NS + sid

    def _zrow(i, carry):
        for r in range(F // 16):
            zb_v[i, pl.ds(r * 16, 16)] = jnp.zeros((16,), jnp.float32)
        return carry

    lax.fori_loop(0, ZR, _zrow, 0)
    zcopies = [
        pltpu.async_copy(zb_v, acc.at[pl.ds(sid * RPT + i * ZR, ZR)],
                         ssems[0])
        for i in range(RPT // ZR)
    ]

    def _make_steps(sbase):

        def _g_start(j, slot):
            pltpu.async_copy(dst_hbm.at[pl.ds(sbase + j * C, C)],
                             dstbs[slot], gsems[slot])
            pltpu.async_copy(x_hbm.at[src_v.at[pl.ds(j * C, C)]],
                             rows_v.at[slot], gsems[slot])

        def _g_wait(j, slot):
            pltpu.make_async_copy(dst_hbm.at[pl.ds(sbase + j * C, C)],
                                  dstbs[slot], gsems[slot]).wait()
            pltpu.make_async_copy(x_hbm.at[src_v.at[pl.ds(j * C, C)]],
                                  rows_v.at[slot], gsems[slot]).wait()

        def _s_start(j, slot):
            pltpu.async_copy(rows_v.at[slot], acc.at[dstbs[slot]],
                             ssems[slot], add=True)

        def _s_wait(j, slot):
            pltpu.make_async_copy(rows_v.at[slot], acc.at[dstbs[slot]],
                                  ssems[slot]).wait()

        def _scale(j, slot):
            def _group(g, carry):
                lvec = lv_v[pl.ds(j * C + g * 16, 16)]
                for t in range(16):
                    e = g * 16 + t
                    sv = jnp.broadcast_to(lvec[t], (16,))
                    for r in range(F // 16):
                        sl = pl.ds(r * 16, 16)
                        rows_v[slot, e, sl] = rows_v[slot, e, sl] * sv
                return carry
            lax.fori_loop(0, C // 16, _group, 0)

        def _process(j, slot, start_next, wait_prev_scatter):
            if start_next:
                if wait_prev_scatter:
                    _s_wait(j - 1, 1 - slot)
                _g_start(j + 1, 1 - slot)
            _g_wait(j, slot)
            if weighted:
                _scale(j, slot)
            _s_start(j, slot)

        return _g_start, _process, _s_wait

    for sb in range(NSB):
        sbase = w * EPW + sb * SBE
        _g_start, _process, _s_wait = _make_steps(sbase)
        pltpu.sync_copy(src_hbm.at[pl.ds(sbase, SBE)], src_v)
        if weighted:
            pltpu.sync_copy(lv_hbm.at[pl.ds(sbase, SBE)], lv_v)
        _g_start(0, 0)
        if sb == 0:
            for zc in zcopies:
                zc.wait()
            plsc.subcore_barrier()

        def _pair(p, carry):
            j = 2 * p + 1
            _process(j, 1, True, True)
            _process(j + 1, 0, True, True)
            return carry

        _process(0, 0, True, False)
        lax.fori_loop(0, (NCHS - 3) // 2, _pair, 0)
        _process(NCHS - 2, 1, True, True)
        _process(NCHS - 1, 0, False, False)
        _s_wait(NCHS - 2, 1)
        _s_wait(NCHS - 1, 0)

    plsc.subcore_barrier()
    pltpu.sync_copy(acc.at[pl.ds(sid * RPT, RPT)],
                    out_hbm.at[cid, pl.ds(sid * RPT, RPT)])


def _make_seg(weighted):
    body = _seg_kernel_weighted if weighted else _seg_kernel_plain
    scratch = [
        pltpu.VMEM((SBE,), jnp.int32),
    ]
    if weighted:
        scratch.append(pltpu.VMEM((SBE,), jnp.float32))
    scratch += [
        pltpu.VMEM((C,), jnp.int32),
        pltpu.VMEM((C,), jnp.int32),
    ]
    scratch += [
        pltpu.VMEM((2, C, F), jnp.float32),
        pltpu.VMEM((ZR, F), jnp.float32),
        pltpu.VMEM_SHARED((VP, F), jnp.float32),
        pltpu.SemaphoreType.DMA,
        pltpu.SemaphoreType.DMA,
        pltpu.SemaphoreType.DMA,
        pltpu.SemaphoreType.DMA,
    ]
    return pl.kernel(
        body,
        out_type=jax.ShapeDtypeStruct((NC, VP, F), jnp.float32),
        mesh=_mesh,
        scratch_types=scratch,
    )


_seg_weighted = _make_seg(True)
_seg_plain = _make_seg(False)


_VB = 400


def _combine1_body(p0, p1, o):
    o[...] = p0[...] + p1[...]


def _combine2_body(p0, p1, x, o):
    o[...] = 2.0 * (p0[...] + p1[...]) - x[...]


def _relu_body(p0, p1, b, o):
    o[...] = jnp.maximum(p0[...] + p1[...] + b[...], 0.0)


def _matmul_body(h, w, o):
    o[...] = jnp.dot(h[...], w[...], preferred_element_type=jnp.float32)


def _ew_spec():
    return pl.BlockSpec((_VB, F), lambda i: (i, 0))


def _combine1(p0, p1):
    return pl.pallas_call(
        _combine1_body,
        grid=(V // _VB,),
        in_specs=[_ew_spec(), _ew_spec()],
        out_specs=_ew_spec(),
        out_shape=jax.ShapeDtypeStruct((V, F), jnp.float32),
    )(p0, p1)


def _combine2(p0, p1, x):
    return pl.pallas_call(
        _combine2_body,
        grid=(V // _VB,),
        in_specs=[_ew_spec(), _ew_spec(), _ew_spec()],
        out_specs=_ew_spec(),
        out_shape=jax.ShapeDtypeStruct((V, F), jnp.float32),
    )(p0, p1, x)


def _relu_out(p0, p1, b2):
    return pl.pallas_call(
        _relu_body,
        grid=(V // _VB,),
        in_specs=[_ew_spec(), _ew_spec(),
                  pl.BlockSpec((1, F), lambda i: (0, 0))],
        out_specs=_ew_spec(),
        out_shape=jax.ShapeDtypeStruct((V, OUT), jnp.float32),
    )(p0, p1, b2)


def _matmul(h, w):
    return pl.pallas_call(
        _matmul_body,
        grid=(V // _VB,),
        in_specs=[pl.BlockSpec((_VB, F * K), lambda i: (i, 0)),
                  pl.BlockSpec((F * K, OUT), lambda i: (0, 0))],
        out_specs=pl.BlockSpec((_VB, OUT), lambda i: (i, 0)),
        out_shape=jax.ShapeDtypeStruct((V, OUT), jnp.float32),
    )(h, w)


def kernel(feature, edge_index, L_values, W, b):
    src = edge_index[0]
    dst = edge_index[1]

    p1 = _seg_weighted(feature, src, dst, L_values)
    t1 = _combine1(p1[0], p1[1])

    p2 = _seg_weighted(t1, src, dst, L_values)
    t2 = _combine2(p2[0], p2[1], feature)

    tst = jnp.stack([feature, t1, t2], axis=0)
    h = jnp.transpose(tst, (2, 0, 1)).reshape(V, F * K)

    hw = _matmul(h, W)

    p3 = _seg_plain(hw, src, dst)
    return _relu_out(p3[0], p3[1], b.reshape(1, OUT))


# speedup vs baseline: 1.3973x; 1.0881x over previous
import functools

import jax
import jax.numpy as jnp
from jax import lax
from jax.experimental import pallas as pl
from jax.experimental.pallas import tpu as pltpu
from jax.experimental.pallas import tpu_sc as plsc

V = 10000
E = 320000
F = 128
K = 3
OUT = 128

NC = 2
NS = 16
NW = NC * NS
EPW = E // NW
C = 80
NCH = EPW // C
NSB = 5
NCHS = NCH // NSB
SBE = NCHS * C
VP = 10240
RPT = VP // NS

_mesh = plsc.VectorSubcoreMesh(core_axis_name="c", subcore_axis_name="s")


def _seg_kernel_weighted(x_hbm, src_hbm, dst_hbm, lv_hbm, out_hbm,
                         src_v, lv_v, db0, db1, db2, rows_v, acc,
                         g0, g1, g2, s0, s1, s2):
    _seg_body(x_hbm, src_hbm, dst_hbm, lv_hbm, out_hbm,
              src_v, lv_v, (db0, db1, db2), rows_v, acc,
              (g0, g1, g2), (s0, s1, s2), weighted=True)


def _seg_kernel_plain(x_hbm, src_hbm, dst_hbm, out_hbm,
                      src_v, db0, db1, db2, rows_v, acc,
                      g0, g1, g2, s0, s1, s2):
    _seg_body(x_hbm, src_hbm, dst_hbm, None, out_hbm,
              src_v, None, (db0, db1, db2), rows_v, acc,
              (g0, g1, g2), (s0, s1, s2), weighted=False)


def _seg_body(x_hbm, src_hbm, dst_hbm, lv_hbm, out_hbm,
              src_v, lv_v, dstbs, rows_v, acc, gsems, ssems, *,
              weighted):
    cid = lax.axis_index("c")
    sid = lax.axis_index("s")
    w = cid * NS + sid

    def _zrow(i, carry):
        for r in range(F // 16):
            rows_v[0, i, pl.ds(r * 16, 16)] = jnp.zeros((16,), jnp.float32)
        return carry

    lax.fori_loop(0, C, _zrow, 0)
    zcopies = [
        pltpu.async_copy(rows_v.at[0], acc.at[pl.ds(sid * RPT + i * C, C)],
                         ssems[0])
        for i in range(RPT // C)
    ]

    def _make_steps(sbase):

        def _g_start(j, slot):
            pltpu.async_copy(dst_hbm.at[pl.ds(sbase + j * C, C)],
                             dstbs[slot], gsems[slot])
            pltpu.async_copy(x_hbm.at[src_v.at[pl.ds(j * C, C)]],
                             rows_v.at[slot], gsems[slot])

        def _g_wait(j, slot):
            pltpu.make_async_copy(dst_hbm.at[pl.ds(sbase + j * C, C)],
                                  dstbs[slot], gsems[slot]).wait()
            pltpu.make_async_copy(x_hbm.at[src_v.at[pl.ds(j * C, C)]],
                                  rows_v.at[slot], gsems[slot]).wait()

        def _s_start(j, slot):
            pltpu.async_copy(rows_v.at[slot], acc.at[dstbs[slot]],
                             ssems[slot], add=True)

        def _s_wait(j, slot):
            pltpu.make_async_copy(rows_v.at[slot], acc.at[dstbs[slot]],
                                  ssems[slot]).wait()

        def _scale(j, slot):
            def _group(g, carry):
                lvec = lv_v[pl.ds(j * C + g * 16, 16)]
                for t in range(16):
                    e = g * 16 + t
                    sv = jnp.broadcast_to(lvec[t], (16,))
                    for r in range(F // 16):
                        sl = pl.ds(r * 16, 16)
                        rows_v[slot, e, sl] = rows_v[slot, e, sl] * sv
                return carry
            lax.fori_loop(0, C // 16, _group, 0)

        def _process(j, slot, start_next, wait_prev_scatter):
            ns = (slot + 1) % 3
            if start_next:
                if wait_prev_scatter:
                    _s_wait(j - 2, ns)
                _g_start(j + 1, ns)
            _g_wait(j, slot)
            if weighted:
                _scale(j, slot)
            _s_start(j, slot)

        return _g_start, _process, _s_wait

    for sb in range(NSB):
        sbase = w * EPW + sb * SBE
        _g_start, _process, _s_wait = _make_steps(sbase)
        pltpu.sync_copy(src_hbm.at[pl.ds(sbase, SBE)], src_v)
        if weighted:
            pltpu.sync_copy(lv_hbm.at[pl.ds(sbase, SBE)], lv_v)
        if sb == 0:
            for zc in zcopies:
                zc.wait()
            plsc.subcore_barrier()
        _g_start(0, 0)

        def _trip(q, carry):
            j = 3 * q + 2
            _process(j, 2, True, True)
            _process(j + 1, 0, True, True)
            _process(j + 2, 1, True, True)
            return carry

        _process(0, 0, True, False)
        _process(1, 1, True, False)
        lax.fori_loop(0, (NCHS - 4) // 3, _trip, 0)
        _process(NCHS - 2, 2, True, True)
        _process(NCHS - 1, 0, False, False)
        _s_wait(NCHS - 3, 1)
        _s_wait(NCHS - 2, 2)
        _s_wait(NCHS - 1, 0)

    plsc.subcore_barrier()
    pltpu.sync_copy(acc.at[pl.ds(sid * RPT, RPT)],
                    out_hbm.at[cid, pl.ds(sid * RPT, RPT)])


def _make_seg(weighted):
    body = _seg_kernel_weighted if weighted else _seg_kernel_plain
    scratch = [
        pltpu.VMEM((SBE,), jnp.int32),
    ]
    if weighted:
        scratch.append(pltpu.VMEM((SBE,), jnp.float32))
    scratch += [
        pltpu.VMEM((C,), jnp.int32),
        pltpu.VMEM((C,), jnp.int32),
        pltpu.VMEM((C,), jnp.int32),
        pltpu.VMEM((3, C, F), jnp.float32),
        pltpu.VMEM_SHARED((VP, F), jnp.float32),
        pltpu.SemaphoreType.DMA,
        pltpu.SemaphoreType.DMA,
        pltpu.SemaphoreType.DMA,
        pltpu.SemaphoreType.DMA,
        pltpu.SemaphoreType.DMA,
        pltpu.SemaphoreType.DMA,
    ]
    return pl.kernel(
        body,
        out_type=jax.ShapeDtypeStruct((NC, VP, F), jnp.float32),
        mesh=_mesh,
        scratch_types=scratch,
    )


_seg_weighted = _make_seg(True)
_seg_plain = _make_seg(False)


_VB = 400


def _combine1_body(p0, p1, o):
    o[...] = p0[...] + p1[...]


def _combine2_body(p0, p1, x, o):
    o[...] = 2.0 * (p0[...] + p1[...]) - x[...]


def _relu_body(p0, p1, b, o):
    o[...] = jnp.maximum(p0[...] + p1[...] + b[...], 0.0)


def _matmul_body(h, w, o):
    o[...] = jnp.dot(h[...], w[...], preferred_element_type=jnp.float32)


def _ew_spec():
    return pl.BlockSpec((_VB, F), lambda i: (i, 0))


def _combine1(p0, p1):
    return pl.pallas_call(
        _combine1_body,
        grid=(V // _VB,),
        in_specs=[_ew_spec(), _ew_spec()],
        out_specs=_ew_spec(),
        out_shape=jax.ShapeDtypeStruct((V, F), jnp.float32),
    )(p0, p1)


def _combine2(p0, p1, x):
    return pl.pallas_call(
        _combine2_body,
        grid=(V // _VB,),
        in_specs=[_ew_spec(), _ew_spec(), _ew_spec()],
        out_specs=_ew_spec(),
        out_shape=jax.ShapeDtypeStruct((V, F), jnp.float32),
    )(p0, p1, x)


def _relu_out(p0, p1, b2):
    return pl.pallas_call(
        _relu_body,
        grid=(V // _VB,),
        in_specs=[_ew_spec(), _ew_spec(),
                  pl.BlockSpec((1, F), lambda i: (0, 0))],
        out_specs=_ew_spec(),
        out_shape=jax.ShapeDtypeStruct((V, OUT), jnp.float32),
    )(p0, p1, b2)


def _matmul(h, w):
    return pl.pallas_call(
        _matmul_body,
        grid=(V // _VB,),
        in_specs=[pl.BlockSpec((_VB, F * K), lambda i: (i, 0)),
                  pl.BlockSpec((F * K, OUT), lambda i: (0, 0))],
        out_specs=pl.BlockSpec((_VB, OUT), lambda i: (i, 0)),
        out_shape=jax.ShapeDtypeStruct((V, OUT), jnp.float32),
    )(h, w)


def kernel(feature, edge_index, L_values, W, b):
    src = edge_index[0]
    dst = edge_index[1]

    p1 = _seg_weighted(feature, src, dst, L_values)
    t1 = _combine1(p1[0], p1[1])

    p2 = _seg_weighted(t1, src, dst, L_values)
    t2 = _combine2(p2[0], p2[1], feature)

    tst = jnp.stack([feature, t1, t2], axis=0)
    h = jnp.transpose(tst, (2, 0, 1)).reshape(V, F * K)

    hw = _matmul(h, W)

    p3 = _seg_plain(hw, src, dst)
    return _relu_out(p3[0], p3[1], b.reshape(1, OUT))


# speedup vs baseline: 1.4802x; 1.0593x over previous
import functools

import jax
import jax.numpy as jnp
from jax import lax
from jax.experimental import pallas as pl
from jax.experimental.pallas import tpu as pltpu
from jax.experimental.pallas import tpu_sc as plsc

V = 10000
E = 320000
F = 128
K = 3
OUT = 128

NC = 2
NS = 16
NW = NC * NS
EPW = E // NW
C = 80
NCH = EPW // C
NSB = 5
NCHS = NCH // NSB
SBE = NCHS * C
VP = 10112
RPT = VP // NS

_mesh = plsc.VectorSubcoreMesh(core_axis_name="c", subcore_axis_name="s")


def _seg_kernel_weighted(x_hbm, src_hbm, dst_hbm, lv_hbm, out_hbm,
                         src_v, lv_v, dst_v, rows_v, acc,
                         g0, g1, g2, s0, s1, s2):
    _seg_body(x_hbm, src_hbm, dst_hbm, lv_hbm, out_hbm,
              src_v, lv_v, dst_v, rows_v, acc,
              (g0, g1, g2), (s0, s1, s2), weighted=True)


def _seg_kernel_plain(x_hbm, src_hbm, dst_hbm, out_hbm,
                      src_v, dst_v, rows_v, acc,
                      g0, g1, g2, s0, s1, s2):
    _seg_body(x_hbm, src_hbm, dst_hbm, None, out_hbm,
              src_v, None, dst_v, rows_v, acc,
              (g0, g1, g2), (s0, s1, s2), weighted=False)


def _seg_body(x_hbm, src_hbm, dst_hbm, lv_hbm, out_hbm,
              src_v, lv_v, dst_v, rows_v, acc, gsems, ssems, *,
              weighted):
    cid = lax.axis_index("c")
    sid = lax.axis_index("s")
    w = cid * NS + sid

    def _zrow(i, carry):
        for r in range(F // 16):
            rows_v[0, i, pl.ds(r * 16, 16)] = jnp.zeros((16,), jnp.float32)
        return carry

    lax.fori_loop(0, C, _zrow, 0)
    zcopies = [
        pltpu.async_copy(rows_v.at[0], acc.at[pl.ds(sid * RPT + i * C, C)],
                         ssems[0])
        for i in range(RPT // C)
    ]
    if RPT % C:
        zcopies.append(pltpu.async_copy(
            rows_v.at[0, pl.ds(0, RPT % C)],
            acc.at[pl.ds(sid * RPT + (RPT // C) * C, RPT % C)], ssems[0]))

    def _make_steps(sbase):

        def _g_start(j, slot):
            pltpu.async_copy(x_hbm.at[src_v.at[pl.ds(j * C, C)]],
                             rows_v.at[slot], gsems[slot])

        def _g_wait(j, slot):
            pltpu.make_async_copy(x_hbm.at[src_v.at[pl.ds(j * C, C)]],
                                  rows_v.at[slot], gsems[slot]).wait()

        def _s_start(j, slot):
            pltpu.async_copy(rows_v.at[slot], acc.at[dst_v.at[j]],
                             ssems[slot], add=True)

        def _s_wait(j, slot):
            pltpu.make_async_copy(rows_v.at[slot], acc.at[dst_v.at[j]],
                                  ssems[slot]).wait()

        def _scale(j, slot):
            def _group(g, carry):
                lvec = lv_v[pl.ds(j * C + g * 16, 16)]
                for t in range(16):
                    e = g * 16 + t
                    sv = jnp.broadcast_to(lvec[t], (16,))
                    for r in range(F // 16):
                        sl = pl.ds(r * 16, 16)
                        rows_v[slot, e, sl] = rows_v[slot, e, sl] * sv
                return carry
            lax.fori_loop(0, C // 16, _group, 0)

        def _process(j, slot, start_next, wait_prev_scatter):
            ns = (slot + 1) % 3
            if start_next:
                if wait_prev_scatter:
                    _s_wait(j - 2, ns)
                _g_start(j + 1, ns)
            _g_wait(j, slot)
            if weighted:
                _scale(j, slot)
            _s_start(j, slot)

        return _g_start, _process, _s_wait

    for sb in range(NSB):
        sbase = w * EPW + sb * SBE
        _g_start, _process, _s_wait = _make_steps(sbase)
        pltpu.sync_copy(src_hbm.at[pl.ds(sbase, SBE)], src_v)
        pltpu.sync_copy(dst_hbm.at[w, sb], dst_v)
        if weighted:
            pltpu.sync_copy(lv_hbm.at[pl.ds(sbase, SBE)], lv_v)
        if sb == 0:
            for zc in zcopies:
                zc.wait()
            plsc.subcore_barrier()
        _g_start(0, 0)

        def _trip(q, carry):
            j = 3 * q + 2
            _process(j, 2, True, True)
            _process(j + 1, 0, True, True)
            _process(j + 2, 1, True, True)
            return carry

        _process(0, 0, True, False)
        _process(1, 1, True, False)
        lax.fori_loop(0, (NCHS - 4) // 3, _trip, 0)
        _process(NCHS - 2, 2, True, True)
        _process(NCHS - 1, 0, False, False)
        _s_wait(NCHS - 3, 1)
        _s_wait(NCHS - 2, 2)
        _s_wait(NCHS - 1, 0)

    plsc.subcore_barrier()
    pltpu.sync_copy(acc.at[pl.ds(sid * RPT, RPT)],
                    out_hbm.at[cid, pl.ds(sid * RPT, RPT)])


def _make_seg(weighted):
    body = _seg_kernel_weighted if weighted else _seg_kernel_plain
    scratch = [
        pltpu.VMEM((SBE,), jnp.int32),
    ]
    if weighted:
        scratch.append(pltpu.VMEM((SBE,), jnp.float32))
    scratch += [
        pltpu.VMEM((NCHS, C), jnp.int32),
        pltpu.VMEM((3, C, F), jnp.float32),
        pltpu.VMEM_SHARED((VP, F), jnp.float32),
        pltpu.SemaphoreType.DMA,
        pltpu.SemaphoreType.DMA,
        pltpu.SemaphoreType.DMA,
        pltpu.SemaphoreType.DMA,
        pltpu.SemaphoreType.DMA,
        pltpu.SemaphoreType.DMA,
    ]
    return pl.kernel(
        body,
        out_type=jax.ShapeDtypeStruct((NC, VP, F), jnp.float32),
        mesh=_mesh,
        scratch_types=scratch,
    )


_seg_weighted = _make_seg(True)
_seg_plain = _make_seg(False)


_VB = 400
_VBE = 1000


def _combine1_body(p0, p1, o):
    o[...] = p0[...] + p1[...]


def _combine2_body(p0, p1, x, o):
    o[...] = 2.0 * (p0[...] + p1[...]) - x[...]


def _relu_body(p0, p1, b, o):
    o[...] = jnp.maximum(p0[...] + p1[...] + b[...], 0.0)


def _matmul_body(h, w, o):
    o[...] = jnp.dot(h[...], w[...], preferred_element_type=jnp.float32)


def _ew_spec():
    return pl.BlockSpec((_VBE, F), lambda i: (i, 0))


def _part_spec(c):
    return pl.BlockSpec((None, _VBE, F), lambda i, c=c: (c, i, 0))


def _combine1(p):
    return pl.pallas_call(
        _combine1_body,
        grid=(V // _VBE,),
        in_specs=[_part_spec(0), _part_spec(1)],
        out_specs=_ew_spec(),
        out_shape=jax.ShapeDtypeStruct((V, F), jnp.float32),
    )(p, p)


def _combine2(p, x):
    return pl.pallas_call(
        _combine2_body,
        grid=(V // _VBE,),
        in_specs=[_part_spec(0), _part_spec(1), _ew_spec()],
        out_specs=_ew_spec(),
        out_shape=jax.ShapeDtypeStruct((V, F), jnp.float32),
    )(p, p, x)


def _relu_out(p, b2):
    return pl.pallas_call(
        _relu_body,
        grid=(V // _VBE,),
        in_specs=[_part_spec(0), _part_spec(1),
                  pl.BlockSpec((1, F), lambda i: (0, 0))],
        out_specs=_ew_spec(),
        out_shape=jax.ShapeDtypeStruct((V, OUT), jnp.float32),
    )(p, p, b2)


def _matmul(h, w):
    return pl.pallas_call(
        _matmul_body,
        grid=(V // _VB,),
        in_specs=[pl.BlockSpec((_VB, F * K), lambda i: (i, 0)),
                  pl.BlockSpec((F * K, OUT), lambda i: (0, 0))],
        out_specs=pl.BlockSpec((_VB, OUT), lambda i: (i, 0)),
        out_shape=jax.ShapeDtypeStruct((V, OUT), jnp.float32),
    )(h, w)


def kernel(feature, edge_index, L_values, W, b):
    src = edge_index[0]
    dst = edge_index[1].reshape(NW, NSB, NCHS, C)

    p1 = _seg_weighted(feature, src, dst, L_values)
    t1 = _combine1(p1)

    p2 = _seg_weighted(t1, src, dst, L_values)
    t2 = _combine2(p2, feature)

    tst = jnp.stack([feature, t1, t2], axis=0)
    h = jnp.transpose(tst, (2, 0, 1)).reshape(V, F * K)

    hw = _matmul(h, W)

    p3 = _seg_plain(hw, src, dst)
    return _relu_out(p3, b.reshape(1, OUT))
